# baseline (speedup 1.0000x reference)
import functools

import jax
import jax.numpy as jnp
from jax import lax
from jax.experimental import pallas as pl
from jax.experimental.pallas import tpu as pltpu

HIDDEN = 1024
N_EXPERTS = 16
TOP_K = 8
MOE_INTER = 1024
SHARED_INTER = 2048
T_TOKENS = 2048


def _router_body(x_ref, gw_ref, comb_ref):
    x = x_ref[...]
    gw = gw_ref[...]
    logits = lax.dot_general(x, gw, (((1,), (1,)), ((), ())),
                             preferred_element_type=jnp.float32)
    m = jnp.max(logits, axis=1, keepdims=True)
    p = jnp.exp(logits - m)
    s = p / jnp.sum(p, axis=1, keepdims=True)
    lane = lax.broadcasted_iota(jnp.int32, s.shape, 1)
    rank = jnp.zeros(s.shape, jnp.int32)
    for ep in range(N_EXPERTS):
        sp = s[:, ep:ep + 1]
        rank = rank + (sp > s).astype(jnp.int32)
        rank = rank + ((sp == s) & (ep < lane)).astype(jnp.int32)
    mask = rank < TOP_K
    w = jnp.where(mask, s, 0.0)
    w = w / jnp.sum(w, axis=1, keepdims=True)
    comb_ref[...] = w


def _shared_body(x_ref, gu_ref, dw_ref, out_ref):
    x = x_ref[...].astype(jnp.bfloat16)
    gu = lax.dot_general(x, gu_ref[...].astype(jnp.bfloat16),
                         (((1,), (1,)), ((), ())),
                         preferred_element_type=jnp.float32)
    g = gu[:, :SHARED_INTER]
    u = gu[:, SHARED_INTER:]
    a = (g * jax.nn.sigmoid(g) * u).astype(jnp.bfloat16)
    out_ref[...] = lax.dot_general(a, dw_ref[...].astype(jnp.bfloat16),
                                   (((1,), (1,)), ((), ())),
                                   preferred_element_type=jnp.float32)


def _experts_body(x_ref, w1_ref, w2_ref, comb_ref, sh_ref, out_ref):
    e = pl.program_id(1)
    x = x_ref[...].astype(jnp.bfloat16)
    h = lax.dot_general(x, w1_ref[0].astype(jnp.bfloat16),
                        (((1,), (1,)), ((), ())),
                        preferred_element_type=jnp.float32)
    h = (h * jax.nn.sigmoid(h)).astype(jnp.bfloat16)
    y = lax.dot_general(h, w2_ref[0].astype(jnp.bfloat16),
                        (((1,), (1,)), ((), ())),
                        preferred_element_type=jnp.float32)
    comb = comb_ref[...]
    lane = lax.broadcasted_iota(jnp.int32, comb.shape, 1)
    w_col = jnp.sum(jnp.where(lane == e, comb, 0.0), axis=1, keepdims=True)
    wy = w_col * y

    @pl.when(e == 0)
    def _():
        out_ref[...] = sh_ref[...] + wy

    @pl.when(e != 0)
    def _():
        out_ref[...] = out_ref[...] + wy


def kernel(hidden_states, gate_w, experts_w1, experts_w2,
           shared_gate_up_w, shared_down_w):
    orig_shape = hidden_states.shape
    x = hidden_states.reshape(-1, orig_shape[-1])
    T = x.shape[0]

    combine = pl.pallas_call(
        _router_body,
        out_shape=jax.ShapeDtypeStruct((T, N_EXPERTS), jnp.float32),
    )(x, gate_w)

    SB = 256
    shared_out = pl.pallas_call(
        _shared_body,
        grid=(T // SB,),
        in_specs=[
            pl.BlockSpec((SB, HIDDEN), lambda t: (t, 0)),
            pl.BlockSpec((2 * SHARED_INTER, HIDDEN), lambda t: (0, 0)),
            pl.BlockSpec((HIDDEN, SHARED_INTER), lambda t: (0, 0)),
        ],
        out_specs=pl.BlockSpec((SB, HIDDEN), lambda t: (t, 0)),
        out_shape=jax.ShapeDtypeStruct((T, HIDDEN), jnp.float32),
    )(x, shared_gate_up_w, shared_down_w)

    TB = 1024
    out = pl.pallas_call(
        _experts_body,
        grid=(T // TB, N_EXPERTS),
        in_specs=[
            pl.BlockSpec((TB, HIDDEN), lambda t, e: (t, 0)),
            pl.BlockSpec((1, MOE_INTER, HIDDEN), lambda t, e: (e, 0, 0)),
            pl.BlockSpec((1, HIDDEN, MOE_INTER), lambda t, e: (e, 0, 0)),
            pl.BlockSpec((TB, N_EXPERTS), lambda t, e: (t, 0)),
            pl.BlockSpec((TB, HIDDEN), lambda t, e: (t, 0)),
        ],
        out_specs=pl.BlockSpec((TB, HIDDEN), lambda t, e: (t, 0)),
        out_shape=jax.ShapeDtypeStruct((T, HIDDEN), jnp.float32),
        compiler_params=pltpu.CompilerParams(
            dimension_semantics=("arbitrary", "arbitrary"),
        ),
    )(x, experts_w1, experts_w2, combine, shared_out)

    return out.reshape(orig_shape)


# speedup vs baseline: 1.0430x; 1.0430x over previous
import functools

import jax
import jax.numpy as jnp
from jax import lax
from jax.experimental import pallas as pl
from jax.experimental.pallas import tpu as pltpu

HIDDEN = 1024
N_EXPERTS = 16
TOP_K = 8
MOE_INTER = 1024
SHARED_INTER = 2048
T_TOKENS = 2048


def _router_body(x_ref, gw_ref, comb_ref):
    x = x_ref[...]
    gw = gw_ref[...]
    logits = lax.dot_general(x, gw, (((1,), (1,)), ((), ())),
                             preferred_element_type=jnp.float32)
    m = jnp.max(logits, axis=1, keepdims=True)
    p = jnp.exp(logits - m)
    s = p / jnp.sum(p, axis=1, keepdims=True)
    lane = lax.broadcasted_iota(jnp.int32, s.shape, 1)
    rank = jnp.zeros(s.shape, jnp.int32)
    for ep in range(N_EXPERTS):
        sp = s[:, ep:ep + 1]
        rank = rank + (sp > s).astype(jnp.int32)
        rank = rank + ((sp == s) & (ep < lane)).astype(jnp.int32)
    mask = rank < TOP_K
    w = jnp.where(mask, s, 0.0)
    w = w / jnp.sum(w, axis=1, keepdims=True)
    comb_ref[...] = w


def _shared_body(x_ref, gu_ref, dw_ref, out_ref):
    x = x_ref[...]
    gu = lax.dot_general(x, gu_ref[...], (((1,), (1,)), ((), ())),
                         preferred_element_type=jnp.float32)
    g = gu[:, :SHARED_INTER]
    u = gu[:, SHARED_INTER:]
    a = g * jax.nn.sigmoid(g) * u
    out_ref[...] = lax.dot_general(a, dw_ref[...], (((1,), (1,)), ((), ())),
                                   preferred_element_type=jnp.float32)


def _experts_body(x_ref, w1_ref, w2_ref, comb_ref, sh_ref, out_ref):
    e = pl.program_id(1)
    x = x_ref[...]
    h = lax.dot_general(x, w1_ref[0], (((1,), (1,)), ((), ())),
                        preferred_element_type=jnp.float32)
    h = h * jax.nn.sigmoid(h)
    y = lax.dot_general(h, w2_ref[0], (((1,), (1,)), ((), ())),
                        preferred_element_type=jnp.float32)
    comb = comb_ref[...]
    lane = lax.broadcasted_iota(jnp.int32, comb.shape, 1)
    w_col = jnp.sum(jnp.where(lane == e, comb, 0.0), axis=1, keepdims=True)
    wy = w_col * y

    @pl.when(e == 0)
    def _():
        out_ref[...] = sh_ref[...] + wy

    @pl.when(e != 0)
    def _():
        out_ref[...] = out_ref[...] + wy


def kernel(hidden_states, gate_w, experts_w1, experts_w2,
           shared_gate_up_w, shared_down_w):
    orig_shape = hidden_states.shape
    x = hidden_states.reshape(-1, orig_shape[-1])
    T = x.shape[0]

    combine = pl.pallas_call(
        _router_body,
        out_shape=jax.ShapeDtypeStruct((T, N_EXPERTS), jnp.float32),
    )(x, gate_w)

    SB = 256
    shared_out = pl.pallas_call(
        _shared_body,
        grid=(T // SB,),
        in_specs=[
            pl.BlockSpec((SB, HIDDEN), lambda t: (t, 0)),
            pl.BlockSpec((2 * SHARED_INTER, HIDDEN), lambda t: (0, 0)),
            pl.BlockSpec((HIDDEN, SHARED_INTER), lambda t: (0, 0)),
        ],
        out_specs=pl.BlockSpec((SB, HIDDEN), lambda t: (t, 0)),
        out_shape=jax.ShapeDtypeStruct((T, HIDDEN), jnp.float32),
    )(x, shared_gate_up_w, shared_down_w)

    TB = 1024
    out = pl.pallas_call(
        _experts_body,
        grid=(T // TB, N_EXPERTS),
        in_specs=[
            pl.BlockSpec((TB, HIDDEN), lambda t, e: (t, 0)),
            pl.BlockSpec((1, MOE_INTER, HIDDEN), lambda t, e: (e, 0, 0)),
            pl.BlockSpec((1, HIDDEN, MOE_INTER), lambda t, e: (e, 0, 0)),
            pl.BlockSpec((TB, N_EXPERTS), lambda t, e: (t, 0)),
            pl.BlockSpec((TB, HIDDEN), lambda t, e: (t, 0)),
        ],
        out_specs=pl.BlockSpec((TB, HIDDEN), lambda t, e: (t, 0)),
        out_shape=jax.ShapeDtypeStruct((T, HIDDEN), jnp.float32),
        compiler_params=pltpu.CompilerParams(
            dimension_semantics=("arbitrary", "arbitrary"),
        ),
    )(x, experts_w1, experts_w2, combine, shared_out)

    return out.reshape(orig_shape)
